# baseline (speedup 1.0000x reference)
import functools

import jax
import jax.numpy as jnp
from jax import lax
from jax.experimental import pallas as pl
from jax.experimental.pallas import tpu as pltpu
from jax.experimental.pallas import tpu_sc as plsc

_B = 1024
_L = 50
_E = 50
_D = 128
_BL = _B * _L

_NC = 2
_NS = 16
_NW = _NC * _NS
_PER_W = _BL // _NW
_CH = 80
_NCH = _PER_W // _CH

_BB = 8
_NEG = -9e15


def _sc_gather_body(emb_hbm, emb2_hbm, idx_hbm, out1_hbm, out2_hbm,
                    idx_v, buf1, buf2, sem1, sem2):
    wid = lax.axis_index("s") * _NC + lax.axis_index("c")
    pltpu.sync_copy(idx_hbm.at[wid], idx_v)
    base = wid * _PER_W

    def chunk(c, carry):
        off = base + c * _CH
        cp1 = pltpu.async_copy(emb_hbm.at[idx_v.at[c]], buf1, sem1)
        cp2 = pltpu.async_copy(emb2_hbm.at[idx_v.at[c]], buf2, sem2)
        cp1.wait()
        pltpu.sync_copy(buf1, out1_hbm.at[pl.ds(off, _CH)])
        cp2.wait()
        pltpu.sync_copy(buf2, out2_hbm.at[pl.ds(off, _CH)])
        return carry

    lax.fori_loop(0, _NCH, chunk, 0)


def _sc_gather2(emb, emb2, idx3):
    mesh = plsc.VectorSubcoreMesh(core_axis_name="c", subcore_axis_name="s")
    fn = pl.kernel(
        _sc_gather_body,
        out_type=(
            jax.ShapeDtypeStruct((_BL, _D), jnp.float32),
            jax.ShapeDtypeStruct((_BL, _D), jnp.float32),
        ),
        mesh=mesh,
        scratch_types=(
            pltpu.VMEM((_NCH, _CH), jnp.int32),
            pltpu.VMEM((_CH, _D), jnp.float32),
            pltpu.VMEM((_CH, _D), jnp.float32),
            pltpu.SemaphoreType.DMA,
            pltpu.SemaphoreType.DMA,
        ),
    )
    return fn(emb, emb2, idx3)


def _attn_body(x_ref, ht_ref, w2_ref, w3_ref, a_ref, a2_ref, ctx_ref, o_ref):
    w2 = w2_ref[...]
    w3 = w3_ref[...]
    a_hi = a_ref[_D:, :]
    a2_lo = a2_ref[:_D, :]
    a2_hi = a2_ref[_D:, :]
    c0 = jnp.sum(ctx_ref[0, :] * a_ref[:_D, 0])

    for i in range(_BB):
        xb = x_ref[i]
        adj = ht_ref[i]
        mask = adj > 0.0
        x4 = jnp.dot(xb, w2, preferred_element_type=jnp.float32)
        s1 = jnp.dot(x4, a_hi, preferred_element_type=jnp.float32)[:, 0] + c0
        s1 = jnp.where(s1 >= 0, s1, 0.2 * s1)
        e1 = jnp.where(mask, s1[None, :], _NEG)
        m1 = jnp.max(e1, axis=1, keepdims=True)
        p1 = jnp.exp(e1 - m1)
        att_edge = p1 / jnp.sum(p1, axis=1, keepdims=True)
        edge = jnp.dot(att_edge, xb, preferred_element_type=jnp.float32)
        edge4 = jnp.dot(edge, w3, preferred_element_type=jnp.float32)
        s2n = jnp.dot(x4, a2_lo, preferred_element_type=jnp.float32)[:, 0]
        s2e = jnp.dot(edge4, a2_hi, preferred_element_type=jnp.float32)[:, 0]
        e2 = s2n[None, :] + s2e[:, None]
        e2 = jnp.where(e2 >= 0, e2, 0.2 * e2)
        att2 = jnp.where(mask, e2, _NEG)
        att2t = att2.T
        m2 = jnp.max(att2t, axis=1, keepdims=True)
        p2 = jnp.exp(att2t - m2)
        att_node = p2 / jnp.sum(p2, axis=1, keepdims=True)
        node = jnp.dot(att_node, edge, preferred_element_type=jnp.float32)
        o_ref[i] = node + xb


def _tc_attention(nodes, HT, w2, w3, a, a2, ctx):
    grid = (_B // _BB,)
    return pl.pallas_call(
        _attn_body,
        grid=grid,
        in_specs=[
            pl.BlockSpec((_BB, _L, _D), lambda i: (i, 0, 0)),
            pl.BlockSpec((_BB, _E, _L), lambda i: (i, 0, 0)),
            pl.BlockSpec((_D, _D), lambda i: (0, 0)),
            pl.BlockSpec((_D, _D), lambda i: (0, 0)),
            pl.BlockSpec((2 * _D, 1), lambda i: (0, 0)),
            pl.BlockSpec((2 * _D, 1), lambda i: (0, 0)),
            pl.BlockSpec((1, _D), lambda i: (0, 0)),
        ],
        out_specs=pl.BlockSpec((_BB, _L, _D), lambda i: (i, 0, 0)),
        out_shape=jax.ShapeDtypeStruct((_B, _L, _D), jnp.float32),
    )(nodes, HT, w2, w3, a, a2, ctx)


def kernel(inputs, HT, G, EG, emb, emb2, w2, w3, a, a2, ctx):
    idx3 = inputs.reshape(_NW, _NCH, _CH).astype(jnp.int32)
    nodes_flat, nodes2_flat = _sc_gather2(emb, emb2, idx3)
    nodes = nodes_flat.reshape(_B, _L, _D)
    nodes2 = nodes2_flat.reshape(_B, _L, _D)
    x = _tc_attention(nodes, HT, w2, w3, a, a2, ctx)
    return (x, x, nodes2)
